# baseline (speedup 1.0000x reference)
import functools

import jax
import jax.numpy as jnp
import numpy as np
from jax import lax
from jax.experimental import pallas as pl
from jax.experimental.pallas import tpu as pltpu
from jax.experimental.pallas import tpu_sc as plsc

N_NODES = 100000
DEG = 64
BATCH = 16384
S = 32
PACK = 4
NPACK = N_NODES // PACK
ROWW = PACK * S

_info = plsc.get_sparse_core_info()
NC, NS, L = _info.num_cores, _info.num_subcores, _info.num_lanes
NW = NC * NS
B_PER_W = BATCH // NW
CHUNK = 128
NCHUNK = B_PER_W // CHUNK
NBAND = S // 8
BANDW = 8 * CHUNK
NG = CHUNK // L

_mesh = plsc.VectorSubcoreMesh(core_axis_name="c", subcore_axis_name="s")


@functools.partial(
    pl.kernel,
    mesh=_mesh,
    compiler_params=pltpu.CompilerParams(
        needs_layout_passes=False, use_tc_tiling_on_sc=False),
    out_type=jax.ShapeDtypeStruct((BATCH * S,), jnp.float32),
    scratch_types=[
        pltpu.VMEM((NCHUNK, CHUNK), jnp.int32),
        pltpu.VMEM((NCHUNK, CHUNK), jnp.int32),
        pltpu.VMEM((CHUNK, ROWW), jnp.float32),
        pltpu.VMEM((CHUNK, ROWW), jnp.float32),
        pltpu.VMEM((S * CHUNK,), jnp.float32),
        pltpu.VMEM((S * CHUNK,), jnp.float32),
        pltpu.VMEM((S,), jnp.float32),
        pltpu.SemaphoreType.DMA,
        pltpu.SemaphoreType.DMA,
        pltpu.SemaphoreType.DMA,
        pltpu.SemaphoreType.DMA,
    ],
)
def _sample_kernel(adj_hbm, ids_hbm, mask_hbm, out_hbm,
                   idx_v, qidx_v, rows0_v, rows1_v, outc0_v, outc1_v, mask_v,
                   gsem0, gsem1, osem0, osem1):
    wid = lax.axis_index("s") * NC + lax.axis_index("c")

    pltpu.sync_copy(ids_hbm.at[pl.ds(wid * NCHUNK, NCHUNK)], idx_v)
    pltpu.sync_copy(mask_hbm, mask_v)
    for c in range(NCHUNK):
        for g in range(NG):
            nv = idx_v[c, pl.ds(g * L, L)]
            qidx_v[c, pl.ds(g * L, L)] = lax.shift_right_logical(nv, 2)

    msplat = [
        plsc.load_gather(mask_v, [jnp.full((L,), j, dtype=jnp.int32)])
        for j in range(S)
    ]
    lanes = lax.iota(jnp.int32, L)

    rows_bufs = (rows0_v, rows1_v)
    out_bufs = (outc0_v, outc1_v)
    gsems = (gsem0, gsem1)
    osems = (osem0, osem1)

    gcopies = [None, None]
    ocopies = [[], []]
    gcopies[0] = pltpu.async_copy(
        adj_hbm.at[qidx_v.at[0]], rows_bufs[0], gsems[0])

    for c in range(NCHUNK):
        rows_v = rows_bufs[c % 2]
        outc_v = out_bufs[c % 2]
        gcopies[c % 2].wait()
        if c + 1 < NCHUNK:
            gcopies[(c + 1) % 2] = pltpu.async_copy(
                adj_hbm.at[qidx_v.at[c + 1]],
                rows_bufs[(c + 1) % 2], gsems[(c + 1) % 2])
        for cp in ocopies[c % 2]:
            cp.wait()

        for g in range(NG):
            nv = idx_v[c, pl.ds(g * L, L)]
            colbase = lax.shift_left(jnp.bitwise_and(nv, 3), 5)
            ridx = jnp.full((L,), g * L, dtype=jnp.int32) + lanes
            for j in range(S):
                vals = plsc.load_gather(rows_v, [ridx, colbase + j])
                outc_v[pl.ds(j * CHUNK + g * L, L)] = vals * msplat[j]

        tile_c = wid * NCHUNK + c
        ocopies[c % 2] = [
            pltpu.async_copy(
                outc_v.at[pl.ds(tr * BANDW, BANDW)],
                out_hbm.at[pl.ds(tr * (BATCH * 8) + tile_c * BANDW, BANDW)],
                osems[c % 2])
            for tr in range(NBAND)
        ]

    for cps in ocopies:
        for cp in cps:
            cp.wait()


_PERM = np.array([
    35, 45, 31, 63, 7, 4, 29, 44, 16, 58, 37, 19, 61, 2, 34, 5,
    30, 42, 3, 39, 56, 22, 6, 54, 18, 10, 11, 53, 32, 15, 49, 50,
    20, 43, 8, 24, 9, 40, 59, 25, 13, 52, 62, 60, 47, 33, 14, 17,
    38, 23, 0, 41, 21, 26, 57, 1, 28, 48, 36, 55, 51, 27, 12, 46,
], dtype=np.int32)


def kernel(adj_info, ids, num_samples, layer):
    del layer
    cols = jnp.asarray(_PERM[:S])
    adj_packed = jnp.take(adj_info, cols, axis=1).reshape(NPACK, ROWW)
    mask = (jnp.arange(S) < num_samples).astype(jnp.float32)
    ids2d = ids.reshape(NW * NCHUNK, CHUNK)
    flat = _sample_kernel(adj_packed, ids2d, mask)
    out = flat.reshape(NBAND, BATCH // CHUNK, 8, CHUNK)
    return out.transpose(1, 3, 0, 2).reshape(BATCH, S)


# speedup vs baseline: 1.3416x; 1.3416x over previous
import functools

import jax
import jax.numpy as jnp
import numpy as np
from jax import lax
from jax.experimental import pallas as pl
from jax.experimental.pallas import tpu as pltpu
from jax.experimental.pallas import tpu_sc as plsc

N_NODES = 100000
DEG = 64
DEGP = 128
BATCH = 16384
S = 32

_info = plsc.get_sparse_core_info()
NC, NS, L = _info.num_cores, _info.num_subcores, _info.num_lanes
NW = NC * NS
B_PER_W = BATCH // NW
CHUNK = 128
NCHUNK = B_PER_W // CHUNK
NBAND = S // 8
BANDW = 8 * CHUNK
NG = CHUNK // L

_mesh = plsc.VectorSubcoreMesh(core_axis_name="c", subcore_axis_name="s")


@functools.partial(
    pl.kernel,
    mesh=_mesh,
    compiler_params=pltpu.CompilerParams(
        needs_layout_passes=False, use_tc_tiling_on_sc=False),
    out_type=jax.ShapeDtypeStruct((BATCH * S,), jnp.float32),
    scratch_types=[
        pltpu.VMEM((NCHUNK, CHUNK), jnp.int32),
        pltpu.VMEM((NCHUNK, CHUNK), jnp.int32),
        pltpu.VMEM((CHUNK, DEG), jnp.float32),
        pltpu.VMEM((CHUNK, DEG), jnp.float32),
        pltpu.VMEM((S * CHUNK,), jnp.float32),
        pltpu.VMEM((S * CHUNK,), jnp.float32),
        pltpu.VMEM((S,), jnp.int32),
        pltpu.VMEM((S,), jnp.float32),
        pltpu.SemaphoreType.DMA,
        pltpu.SemaphoreType.DMA,
        pltpu.SemaphoreType.DMA,
        pltpu.SemaphoreType.DMA,
    ],
)
def _sample_kernel(adj_hbm, ids_hbm, cols_hbm, mask_hbm, out_hbm,
                   idx_v, qidx_v, rows0_v, rows1_v, outc0_v, outc1_v,
                   cols_v, mask_v, gsem0, gsem1, osem0, osem1):
    wid = lax.axis_index("s") * NC + lax.axis_index("c")

    pltpu.sync_copy(ids_hbm.at[pl.ds(wid * NCHUNK, NCHUNK)], idx_v)
    pltpu.sync_copy(cols_hbm, cols_v)
    pltpu.sync_copy(mask_hbm, mask_v)
    for c in range(NCHUNK):
        for g in range(NG):
            nv = idx_v[c, pl.ds(g * L, L)]
            qidx_v[c, pl.ds(g * L, L)] = lax.shift_left(nv, 1)

    cols_lo = cols_v[pl.ds(0, L)]
    cols_hi = cols_v[pl.ds(L, L)]
    m_lo = mask_v[pl.ds(0, L)]
    m_hi = mask_v[pl.ds(L, L)]
    off_lo = lax.iota(jnp.int32, L) * CHUNK
    off_hi = off_lo + L * CHUNK

    rows_bufs = (rows0_v, rows1_v)
    out_bufs = (outc0_v, outc1_v)
    gsems = (gsem0, gsem1)
    osems = (osem0, osem1)

    gcopies = [None, None]
    ocopies = [[], []]
    gcopies[0] = pltpu.async_copy(
        adj_hbm.at[qidx_v.at[0]], rows_bufs[0], gsems[0])

    for c in range(NCHUNK):
        rows_v = rows_bufs[c % 2]
        outc_v = out_bufs[c % 2]
        gcopies[c % 2].wait()
        if c + 1 < NCHUNK:
            gcopies[(c + 1) % 2] = pltpu.async_copy(
                adj_hbm.at[qidx_v.at[c + 1]],
                rows_bufs[(c + 1) % 2], gsems[(c + 1) % 2])
        for cp in ocopies[c % 2]:
            cp.wait()

        @plsc.parallel_loop(0, CHUNK, unroll=4)
        def body(r):
            ridx = jnp.full((L,), r, dtype=jnp.int32)
            lo = plsc.load_gather(rows_v, [ridx, cols_lo]) * m_lo
            hi = plsc.load_gather(rows_v, [ridx, cols_hi]) * m_hi
            plsc.store_scatter(outc_v, [ridx + off_lo], lo)
            plsc.store_scatter(outc_v, [ridx + off_hi], hi)

        tile_c = wid * NCHUNK + c
        ocopies[c % 2] = [
            pltpu.async_copy(
                outc_v.at[pl.ds(tr * BANDW, BANDW)],
                out_hbm.at[pl.ds(tr * (BATCH * 8) + tile_c * BANDW, BANDW)],
                osems[c % 2])
            for tr in range(NBAND)
        ]

    for cps in ocopies:
        for cp in cps:
            cp.wait()


_PERM = np.array([
    35, 45, 31, 63, 7, 4, 29, 44, 16, 58, 37, 19, 61, 2, 34, 5,
    30, 42, 3, 39, 56, 22, 6, 54, 18, 10, 11, 53, 32, 15, 49, 50,
    20, 43, 8, 24, 9, 40, 59, 25, 13, 52, 62, 60, 47, 33, 14, 17,
    38, 23, 0, 41, 21, 26, 57, 1, 28, 48, 36, 55, 51, 27, 12, 46,
], dtype=np.int32)


def kernel(adj_info, ids, num_samples, layer):
    del layer
    adj_pad = jnp.pad(adj_info, ((0, 0), (0, DEGP - DEG)))
    adj2 = adj_pad.reshape(2 * N_NODES, DEG)
    cols = jnp.asarray(_PERM[:S])
    mask = (jnp.arange(S) < num_samples).astype(jnp.float32)
    ids2d = ids.reshape(NW * NCHUNK, CHUNK)
    flat = _sample_kernel(adj2, ids2d, cols, mask)
    out = flat.reshape(NBAND, BATCH // CHUNK, 8, CHUNK)
    return out.transpose(1, 3, 0, 2).reshape(BATCH, S)
